# Initial kernel scaffold; baseline (speedup 1.0000x reference)
import functools

import jax
import jax.numpy as jnp
from jax import lax
from jax.experimental import pallas as pl
from jax.experimental.pallas import tpu as pltpu
from jax.experimental.pallas import tpu_sc as plsc

_N = 10000
_E = 320000
_H = 128
_B = 10
_NPG = 1000
_ZA = 64
_ZI = 64

_NC = 2
_NS = 16
_NW = _NC * _NS
_EPW = _E // _NW
_K = 80
_NCH = _EPW // _K
_RPS = _N // _NS

_F32 = jnp.float32


def _mesh():
    return plsc.VectorSubcoreMesh(core_axis_name="c", subcore_axis_name="s")


def _sc_hist(dst_r):
    def body(dst_hbm, out_hbm, histv, idxv):
        cid = lax.axis_index("c")
        sid = lax.axis_index("s")
        wid = cid * _NS + sid
        pltpu.sync_copy(dst_hbm.at[wid], idxv)
        zero16 = jnp.zeros((16,), _F32)
        one16 = jnp.ones((16,), _F32)

        def zstep(j, carry):
            histv[pl.ds(j * 16, 16)] = zero16
            return carry

        lax.fori_loop(0, _N // 16, zstep, 0)

        def astep(j, carry):
            plsc.addupdate_scatter(histv, [idxv[j]], one16)
            return carry

        lax.fori_loop(0, _EPW // 16, astep, 0)
        pltpu.sync_copy(histv, out_hbm.at[wid])

    return pl.kernel(
        body,
        out_type=jax.ShapeDtypeStruct((_NW, _N), _F32),
        mesh=_mesh(),
        scratch_types=[
            pltpu.VMEM((_N,), _F32),
            pltpu.VMEM((_EPW // 16, 16), jnp.int32),
        ],
    )(dst_r)


def _sc_mp(src_r, dst_r, u, zrows):
    def body(src_hbm, dst_hbm, u_hbm, z_hbm, out_hbm, idxs, idxd, rows, acc, sem):
        cid = lax.axis_index("c")
        sid = lax.axis_index("s")
        wid = cid * _NS + sid
        row0 = sid * _RPS
        pltpu.sync_copy(z_hbm.at[pl.ds(row0, _RPS)], acc.at[pl.ds(row0, _RPS)])
        pltpu.sync_copy(src_hbm.at[wid], idxs)
        pltpu.sync_copy(dst_hbm.at[wid], idxd)
        plsc.subcore_barrier()

        def chunk(i, carry):
            pltpu.async_copy(u_hbm.at[idxs.at[i]], rows, sem).wait()
            pltpu.sync_copy(rows, acc.at[idxd.at[i]], add=True)
            return carry

        lax.fori_loop(0, _NCH, chunk, 0)
        plsc.subcore_barrier()
        pltpu.sync_copy(acc.at[pl.ds(row0, _RPS)],
                        out_hbm.at[pl.ds(cid * _N + row0, _RPS)])

    return pl.kernel(
        body,
        out_type=jax.ShapeDtypeStruct((_NC * _N, _H), _F32),
        mesh=_mesh(),
        scratch_types=[
            pltpu.VMEM((_NCH, _K), jnp.int32),
            pltpu.VMEM((_NCH, _K), jnp.int32),
            pltpu.VMEM((_K, _H), _F32),
            pltpu.VMEM_SHARED((_N, _H), _F32),
            pltpu.SemaphoreType.DMA,
        ],
    )(src_r, dst_r, u, zrows)


def _tc_deg(hist):
    def body(h_ref, o_ref):
        ones = jnp.ones((_NW, 1), _F32)
        s = lax.dot_general(h_ref[...], ones, (((0,), (0,)), ((), ())),
                            preferred_element_type=_F32)
        o_ref[...] = lax.rsqrt(s + 1.0)

    return pl.pallas_call(
        body, out_shape=jax.ShapeDtypeStruct((_N, 1), _F32))(hist)


def _tc_u1(x, W, dinv):
    def body(x_ref, w_ref, d_ref, o_ref):
        o_ref[...] = jnp.dot(x_ref[...], w_ref[...],
                             preferred_element_type=_F32) * d_ref[...]

    return pl.pallas_call(
        body, out_shape=jax.ShapeDtypeStruct((_N, _H), _F32))(x, W, dinv)


def _tc_layer(acc, u, dinv, b, g, be, Wn):
    def body(a_ref, u_ref, d_ref, b_ref, g_ref, be_ref, *rest):
        if Wn is None:
            (h_ref,) = rest
        else:
            w_ref, h_ref, un_ref = rest
        s = a_ref[0:_N, :] + a_ref[_N:2 * _N, :] + u_ref[...]
        pre = s * d_ref[...] + b_ref[...]
        m = jnp.mean(pre, axis=0)
        v = jnp.mean(pre * pre, axis=0) - m * m
        hl = jnp.maximum((pre - m) / jnp.sqrt(v + 1e-5) * g_ref[...]
                         + be_ref[...], 0.0)
        h_ref[...] = hl
        if Wn is not None:
            un_ref[...] = jnp.dot(hl, w_ref[...],
                                  preferred_element_type=_F32) * d_ref[...]

    if Wn is None:
        out_shape = jax.ShapeDtypeStruct((_N, _H), _F32)
        return pl.pallas_call(body, out_shape=out_shape)(acc, u, dinv, b, g, be)
    out_shape = (jax.ShapeDtypeStruct((_N, _H), _F32),
                 jax.ShapeDtypeStruct((_N, _H), _F32))
    return pl.pallas_call(body, out_shape=out_shape)(acc, u, dinv, b, g, be, Wn)


def _tc_pool(h1, h2, h3, Wp1, bp1, Wp2):
    def body(h1_ref, h2_ref, h3_ref, w1_ref, b1_ref, w2_ref, o_ref):
        phi = (jnp.dot(h1_ref[...], w1_ref[...], preferred_element_type=_F32)
               + b1_ref[...]
               + jnp.dot(h2_ref[...], w2_ref[...], preferred_element_type=_F32))
        m = jnp.max(phi)
        e = jnp.exp(phi - m)
        a = e / jnp.sum(e)
        o_ref[...] = lax.dot_general(a, h3_ref[...], (((0,), (0,)), ((), ())),
                                     preferred_element_type=_F32)

    seg = pl.BlockSpec((_NPG, _H), lambda i: (i, 0))
    rep2 = pl.BlockSpec((_H, 1), lambda i: (0, 0))
    rep1 = pl.BlockSpec((1, 1), lambda i: (0, 0))
    return pl.pallas_call(
        body,
        grid=(_B,),
        in_specs=[seg, seg, seg, rep2, rep1, rep2],
        out_specs=pl.BlockSpec((1, _H), lambda i: (i, 0)),
        out_shape=jax.ShapeDtypeStruct((_B, _H), _F32),
    )(h1, h2, h3, Wp1, bp1, Wp2)


def _tc_head_a(hg, WmuA, bmuA, WlvA, blvA, eps_A, WAt):
    def body(hg_ref, wm_ref, bm_ref, wl_ref, bl_ref, e_ref, wa_ref,
             mu_ref, lv_ref, z_ref, zt_ref):
        mu = jnp.dot(hg_ref[...], wm_ref[...],
                     preferred_element_type=_F32) + bm_ref[...]
        lv = jnp.dot(hg_ref[...], wl_ref[...],
                     preferred_element_type=_F32) + bl_ref[...]
        z = mu + e_ref[...] * jnp.exp(0.5 * lv)
        mu_ref[...] = mu
        lv_ref[...] = lv
        z_ref[...] = z
        zt_ref[...] = jnp.dot(z, wa_ref[...], preferred_element_type=_F32)

    out_shape = (jax.ShapeDtypeStruct((_B, _ZA), _F32),
                 jax.ShapeDtypeStruct((_B, _ZA), _F32),
                 jax.ShapeDtypeStruct((_B, _ZA), _F32),
                 jax.ShapeDtypeStruct((_B, _NPG * _ZI), _F32))
    return pl.pallas_call(body, out_shape=out_shape)(
        hg, WmuA, bmuA, WlvA, blvA, eps_A, WAt)


def _tc_head_b(h3, zt2, Wh, bh, Wmui, bmui, Wlvi, blvi, eps_i):
    def body(h3_ref, zt_ref, wh_ref, bh_ref, wm_ref, bm_ref, wl_ref, bl_ref,
             e_ref, zi_ref, mu_ref, lv_ref):
        s = (jnp.dot(h3_ref[...], wh_ref[...], preferred_element_type=_F32)
             + bh_ref[...] + zt_ref[...])
        mu = jnp.dot(s, wm_ref[...], preferred_element_type=_F32) + bm_ref[...]
        lv = jnp.dot(s, wl_ref[...], preferred_element_type=_F32) + bl_ref[...]
        zi_ref[...] = mu + e_ref[...] * jnp.exp(0.5 * lv)
        mu_ref[...] = mu
        lv_ref[...] = lv

    out_shape = (jax.ShapeDtypeStruct((_N, _ZI), _F32),
                 jax.ShapeDtypeStruct((_N, _ZI), _F32),
                 jax.ShapeDtypeStruct((_N, _ZI), _F32))
    return pl.pallas_call(body, out_shape=out_shape)(
        h3, zt2, Wh, bh, Wmui, bmui, Wlvi, blvi, eps_i)


def kernel(x, edge_index, batch, W1, b1, W2, b2, W3, b3, g1, be1, g2, be2,
           g3, be3, Wp1, bp1, Wp2, WmuA, bmuA, WlvA, blvA, Wh, bh, W_A,
           Wmui, bmui, Wlvi, blvi):
    src = edge_index[0]
    dst = edge_index[1]
    src_r = src.reshape(_NW, _NCH, _K)
    dst_r = dst.reshape(_NW, _NCH, _K)
    dst_h = dst.reshape(_NW, _EPW // 16, 16)
    zrows = jnp.zeros((_N, _H), _F32)

    hist = _sc_hist(dst_h)
    dinv = _tc_deg(hist)

    u1 = _tc_u1(x, W1, dinv)
    acc1 = _sc_mp(src_r, dst_r, u1, zrows)
    h1, u2 = _tc_layer(acc1, u1, dinv, b1, g1, be1, W2)
    acc2 = _sc_mp(src_r, dst_r, u2, zrows)
    h2, u3 = _tc_layer(acc2, u2, dinv, b2, g2, be2, W3)
    acc3 = _sc_mp(src_r, dst_r, u3, zrows)
    h3 = _tc_layer(acc3, u3, dinv, b3, g3, be3, None)

    hg = _tc_pool(h1, h2, h3, Wp1, bp1.reshape(1, 1), Wp2)

    rk = jax.random.key(42)
    eps_A = jax.random.normal(jax.random.fold_in(rk, 1), (_B, _ZA), _F32)
    eps_i = jax.random.normal(jax.random.fold_in(rk, 2), (_N, _ZI), _F32)

    WAt = jnp.transpose(W_A, (1, 0, 2)).reshape(_ZA, _NPG * _ZI)
    mu_A, logvar_A, z_A, zt = _tc_head_a(hg, WmuA, bmuA, WlvA, blvA,
                                         eps_A, WAt)
    zt2 = zt.reshape(_N, _ZI)
    z_i, mu_i, logvar_i = _tc_head_b(h3, zt2, Wh, bh, Wmui, bmui,
                                     Wlvi, blvi, eps_i)
    return (z_i, mu_i, logvar_i, z_A, mu_A, logvar_A)


# speedup vs baseline: 15.4586x; 15.4586x over previous
import functools

import jax
import jax.numpy as jnp
from jax import lax
from jax.experimental import pallas as pl
from jax.experimental.pallas import tpu as pltpu
from jax.experimental.pallas import tpu_sc as plsc

_N = 10000
_E = 320000
_H = 128
_B = 10
_NPG = 1000
_ZA = 64
_ZI = 64

_NC = 2
_NS = 16
_NW = _NC * _NS
_EPW = _E // _NW
_K = 80
_NCH = _EPW // _K
_NP = 10240
_RPS = _NP // _NS

_F32 = jnp.float32


def _mesh():
    return plsc.VectorSubcoreMesh(core_axis_name="c", subcore_axis_name="s")


def _sc_deg(dst_r, ones_k, zcol):
    def body(dst_hbm, ones_hbm, z_hbm, out_hbm, idxd, onesv, accd):
        cid = lax.axis_index("c")
        sid = lax.axis_index("s")
        wid = cid * _NS + sid
        pltpu.sync_copy(dst_hbm.at[wid], idxd)
        pltpu.sync_copy(ones_hbm, onesv)

        @pl.when(sid == 0)
        def _():
            pltpu.sync_copy(z_hbm, accd)

        plsc.subcore_barrier()

        def chunk(i, carry):
            pltpu.sync_copy(onesv, accd.at[idxd.at[i]], add=True)
            return carry

        lax.fori_loop(0, _NCH, chunk, 0)
        plsc.subcore_barrier()

        @pl.when(sid == 0)
        def _():
            pltpu.sync_copy(accd, out_hbm.at[cid])

    return pl.kernel(
        body,
        out_type=jax.ShapeDtypeStruct((_NC, _N), _F32),
        mesh=_mesh(),
        scratch_types=[
            pltpu.VMEM((_NCH, _K), jnp.int32),
            pltpu.VMEM((_K,), _F32),
            pltpu.VMEM_SHARED((_N,), _F32),
        ],
    )(dst_r, ones_k, zcol)


def _sc_mp(src_r, dst_r, u, zrows):
    def body(src_hbm, dst_hbm, u_hbm, z_hbm, out_hbm, idxs, idxd, rows, acc, sem):
        cid = lax.axis_index("c")
        sid = lax.axis_index("s")
        wid = cid * _NS + sid
        row0 = sid * _RPS
        pltpu.sync_copy(z_hbm.at[pl.ds(row0, _RPS)], acc.at[pl.ds(row0, _RPS)])
        pltpu.sync_copy(src_hbm.at[wid], idxs)
        pltpu.sync_copy(dst_hbm.at[wid], idxd)
        plsc.subcore_barrier()

        def chunk(i, carry):
            pltpu.async_copy(u_hbm.at[idxs.at[i]], rows, sem).wait()
            pltpu.sync_copy(rows, acc.at[idxd.at[i]], add=True)
            return carry

        lax.fori_loop(0, _NCH, chunk, 0)
        plsc.subcore_barrier()
        pltpu.sync_copy(acc.at[pl.ds(row0, _RPS)],
                        out_hbm.at[pl.ds(cid * _NP + row0, _RPS)])

    return pl.kernel(
        body,
        out_type=jax.ShapeDtypeStruct((_NC * _NP, _H), _F32),
        mesh=_mesh(),
        scratch_types=[
            pltpu.VMEM((_NCH, _K), jnp.int32),
            pltpu.VMEM((_NCH, _K), jnp.int32),
            pltpu.VMEM((_K, _H), _F32),
            pltpu.VMEM_SHARED((_NP, _H), _F32),
            pltpu.SemaphoreType.DMA,
        ],
    )(src_r, dst_r, u, zrows)


def _tc_deg(hist):
    def body(h_ref, o_ref):
        ones = jnp.ones((_NC, 1), _F32)
        s = lax.dot_general(h_ref[...], ones, (((0,), (0,)), ((), ())),
                            preferred_element_type=_F32)
        o_ref[...] = lax.rsqrt(s + 1.0)

    return pl.pallas_call(
        body, out_shape=jax.ShapeDtypeStruct((_N, 1), _F32))(hist)


def _tc_u1(x, W, dinv):
    def body(x_ref, w_ref, d_ref, o_ref):
        o_ref[...] = jnp.dot(x_ref[...], w_ref[...],
                             preferred_element_type=_F32) * d_ref[...]

    return pl.pallas_call(
        body, out_shape=jax.ShapeDtypeStruct((_N, _H), _F32))(x, W, dinv)


def _tc_layer(acc, u, dinv, b, g, be, Wn):
    def body(a_ref, u_ref, d_ref, b_ref, g_ref, be_ref, *rest):
        if Wn is None:
            (h_ref,) = rest
        else:
            w_ref, h_ref, un_ref = rest
        s = a_ref[0:_N, :] + a_ref[_NP:_NP + _N, :] + u_ref[...]
        pre = s * d_ref[...] + b_ref[...]
        m = jnp.mean(pre, axis=0)
        v = jnp.mean(pre * pre, axis=0) - m * m
        hl = jnp.maximum((pre - m) / jnp.sqrt(v + 1e-5) * g_ref[...]
                         + be_ref[...], 0.0)
        h_ref[...] = hl
        if Wn is not None:
            un_ref[...] = jnp.dot(hl, w_ref[...],
                                  preferred_element_type=_F32) * d_ref[...]

    if Wn is None:
        out_shape = jax.ShapeDtypeStruct((_N, _H), _F32)
        return pl.pallas_call(body, out_shape=out_shape)(acc, u, dinv, b, g, be)
    out_shape = (jax.ShapeDtypeStruct((_N, _H), _F32),
                 jax.ShapeDtypeStruct((_N, _H), _F32))
    return pl.pallas_call(body, out_shape=out_shape)(acc, u, dinv, b, g, be, Wn)


def _tc_pool(h1, h2, h3, Wp1, bp1, Wp2):
    def body(h1_ref, h2_ref, h3_ref, w1_ref, b1_ref, w2_ref, o_ref):
        phi = (jnp.dot(h1_ref[...], w1_ref[...], preferred_element_type=_F32)
               + b1_ref[...]
               + jnp.dot(h2_ref[...], w2_ref[...], preferred_element_type=_F32))
        m = jnp.max(phi)
        e = jnp.exp(phi - m)
        a = e / jnp.sum(e)
        hg = lax.dot_general(a, h3_ref[...], (((0,), (0,)), ((), ())),
                             preferred_element_type=_F32)
        o_ref[...] = hg.reshape(1, 1, _H)

    seg = pl.BlockSpec((_NPG, _H), lambda i: (i, 0))
    rep2 = pl.BlockSpec((_H, 1), lambda i: (0, 0))
    rep1 = pl.BlockSpec((1, 1), lambda i: (0, 0))
    out = pl.pallas_call(
        body,
        grid=(_B,),
        in_specs=[seg, seg, seg, rep2, rep1, rep2],
        out_specs=pl.BlockSpec((1, 1, _H), lambda i: (i, 0, 0)),
        out_shape=jax.ShapeDtypeStruct((_B, 1, _H), _F32),
    )(h1, h2, h3, Wp1, bp1, Wp2)
    return out.reshape(_B, _H)


def _tc_head_a(hg, WmuA, bmuA, WlvA, blvA, eps_A, WAt):
    def body(hg_ref, wm_ref, bm_ref, wl_ref, bl_ref, e_ref, wa_ref,
             mu_ref, lv_ref, z_ref, zt_ref):
        mu = jnp.dot(hg_ref[...], wm_ref[...],
                     preferred_element_type=_F32) + bm_ref[...]
        lv = jnp.dot(hg_ref[...], wl_ref[...],
                     preferred_element_type=_F32) + bl_ref[...]
        z = mu + e_ref[...] * jnp.exp(0.5 * lv)
        mu_ref[...] = mu
        lv_ref[...] = lv
        z_ref[...] = z
        zt_ref[...] = jnp.dot(z, wa_ref[...], preferred_element_type=_F32)

    out_shape = (jax.ShapeDtypeStruct((_B, _ZA), _F32),
                 jax.ShapeDtypeStruct((_B, _ZA), _F32),
                 jax.ShapeDtypeStruct((_B, _ZA), _F32),
                 jax.ShapeDtypeStruct((_B, _NPG * _ZI), _F32))
    return pl.pallas_call(body, out_shape=out_shape)(
        hg, WmuA, bmuA, WlvA, blvA, eps_A, WAt)


def _tc_head_b(h3, zt2, Wh, bh, Wmui, bmui, Wlvi, blvi, eps_i):
    def body(h3_ref, zt_ref, wh_ref, bh_ref, wm_ref, bm_ref, wl_ref, bl_ref,
             e_ref, zi_ref, mu_ref, lv_ref):
        s = (jnp.dot(h3_ref[...], wh_ref[...], preferred_element_type=_F32)
             + bh_ref[...] + zt_ref[...])
        mu = jnp.dot(s, wm_ref[...], preferred_element_type=_F32) + bm_ref[...]
        lv = jnp.dot(s, wl_ref[...], preferred_element_type=_F32) + bl_ref[...]
        zi_ref[...] = mu + e_ref[...] * jnp.exp(0.5 * lv)
        mu_ref[...] = mu
        lv_ref[...] = lv

    out_shape = (jax.ShapeDtypeStruct((_N, _ZI), _F32),
                 jax.ShapeDtypeStruct((_N, _ZI), _F32),
                 jax.ShapeDtypeStruct((_N, _ZI), _F32))
    return pl.pallas_call(body, out_shape=out_shape)(
        h3, zt2, Wh, bh, Wmui, bmui, Wlvi, blvi, eps_i)


def kernel(x, edge_index, batch, W1, b1, W2, b2, W3, b3, g1, be1, g2, be2,
           g3, be3, Wp1, bp1, Wp2, WmuA, bmuA, WlvA, blvA, Wh, bh, W_A,
           Wmui, bmui, Wlvi, blvi):
    src = edge_index[0]
    dst = edge_index[1]
    src_r = src.reshape(_NW, _NCH, _K)
    dst_r = dst.reshape(_NW, _NCH, _K)
    zrows = jnp.zeros((_NP, _H), _F32)

    hist = _sc_deg(dst_r, jnp.ones((_K,), _F32), jnp.zeros((_N,), _F32))
    dinv = _tc_deg(hist)

    u1 = _tc_u1(x, W1, dinv)
    acc1 = _sc_mp(src_r, dst_r, u1, zrows)
    h1, u2 = _tc_layer(acc1, u1, dinv, b1, g1, be1, W2)
    acc2 = _sc_mp(src_r, dst_r, u2, zrows)
    h2, u3 = _tc_layer(acc2, u2, dinv, b2, g2, be2, W3)
    acc3 = _sc_mp(src_r, dst_r, u3, zrows)
    h3 = _tc_layer(acc3, u3, dinv, b3, g3, be3, None)

    hg = _tc_pool(h1, h2, h3, Wp1, bp1.reshape(1, 1), Wp2)

    rk = jax.random.key(42)
    eps_A = jax.random.normal(jax.random.fold_in(rk, 1), (_B, _ZA), _F32)
    eps_i = jax.random.normal(jax.random.fold_in(rk, 2), (_N, _ZI), _F32)

    WAt = jnp.transpose(W_A, (1, 0, 2)).reshape(_ZA, _NPG * _ZI)
    mu_A, logvar_A, z_A, zt = _tc_head_a(hg, WmuA, bmuA, WlvA, blvA,
                                         eps_A, WAt)
    zt2 = zt.reshape(_N, _ZI)
    z_i, mu_i, logvar_i = _tc_head_b(h3, zt2, Wh, bh, Wmui, bmui,
                                     Wlvi, blvi, eps_i)
    return (z_i, mu_i, logvar_i, z_A, mu_A, logvar_A)
